# baseline (speedup 1.0000x reference)
import dataclasses

import jax
import jax.numpy as jnp
from jax import lax
from jax.experimental import pallas as pl
from jax.experimental.pallas import tpu as pltpu
from jax.experimental.pallas import tpu_sc as plsc

BATCH = 16384
RANK = 32
NC = 2
NS = 16
LANES = 16
NW = NC * NS
BPW = BATCH // NW
CHUNK = 128
NCH = BPW // CHUNK


def _dot_gather_body(uidx_hbm, vidx_hbm, u_hbm, v_hbm, out_hbm,
                     idx_u, idx_v, rows_u, rows_v, out_v, sem_u, sem_v):
    wid = lax.axis_index("s") * NC + lax.axis_index("c")
    ibase = wid * NCH

    pltpu.sync_copy(uidx_hbm.at[pl.ds(ibase, NCH)], idx_u)
    pltpu.sync_copy(vidx_hbm.at[pl.ds(ibase, NCH)], idx_v)

    copies = []
    for j in range(NCH):
        cu = pltpu.async_copy(
            u_hbm.at[idx_u.at[j]], rows_u.at[pl.ds(j * CHUNK, CHUNK)],
            sem_u.at[j])
        cv = pltpu.async_copy(
            v_hbm.at[idx_v.at[j]], rows_v.at[pl.ds(j * CHUNK, CHUNK)],
            sem_v.at[j])
        copies.append((cu, cv))

    lane_iota = lax.iota(jnp.int32, LANES)

    for j, (cu, cv) in enumerate(copies):
        cu.wait()
        cv.wait()

        @pl.loop(0, CHUNK // LANES)
        def _(g, j=j):
            rows16 = (j * CHUNK + g * LANES) + lane_iota
            acc = jnp.zeros((LANES,), jnp.float32)
            for d in range(RANK):
                dcol = jnp.full((LANES,), d, jnp.int32)
                uu = plsc.load_gather(rows_u, [rows16, dcol])
                vv = plsc.load_gather(rows_v, [rows16, dcol])
                acc = acc + uu * vv
            out_v[pl.ds(j * CHUNK + g * LANES, LANES)] = acc

    pltpu.sync_copy(out_v, out_hbm.at[pl.ds(wid * BPW, BPW)])


def kernel(idxs, U, V):
    idxs = idxs.astype(jnp.int32)
    uidx = idxs[:, 0].reshape(NW * NCH, CHUNK)
    vidx = idxs[:, 1].reshape(NW * NCH, CHUNK)
    mesh = plsc.VectorSubcoreMesh(core_axis_name="c", subcore_axis_name="s")
    cp = pltpu.CompilerParams()
    if "needs_layout_passes" in pltpu.CompilerParams.__dataclass_fields__:
        cp = dataclasses.replace(cp, needs_layout_passes=False)
    if "use_tc_tiling_on_sc" in pltpu.CompilerParams.__dataclass_fields__:
        cp = dataclasses.replace(cp, use_tc_tiling_on_sc=False)
    run = pl.kernel(
        _dot_gather_body,
        out_type=jax.ShapeDtypeStruct((BATCH,), jnp.float32),
        mesh=mesh,
        scratch_types=[
            pltpu.VMEM((NCH, CHUNK), jnp.int32),
            pltpu.VMEM((NCH, CHUNK), jnp.int32),
            pltpu.VMEM((BPW, RANK), jnp.float32),
            pltpu.VMEM((BPW, RANK), jnp.float32),
            pltpu.VMEM((BPW,), jnp.float32),
            pltpu.SemaphoreType.DMA((NCH,)),
            pltpu.SemaphoreType.DMA((NCH,)),
        ],
        compiler_params=cp,
    )
    return run(uidx, vidx, U, V)


# speedup vs baseline: 3.3614x; 3.3614x over previous
import dataclasses

import jax
import jax.numpy as jnp
from jax import lax
from jax.experimental import pallas as pl
from jax.experimental.pallas import tpu as pltpu
from jax.experimental.pallas import tpu_sc as plsc

BATCH = 16384
RANK = 32
NC = 2
NS = 16
LANES = 16
NW = NC * NS
BPW = BATCH // NW
NGRP = BPW // LANES
BLK = 128


def _gather_pass(t_hbm, idx_ref, cols_ref, blk_ref, sem, lane_iota):
    def fire(g):
        i16 = idx_ref[pl.ds(g * LANES, LANES)]
        for k in range(LANES):
            c = pl.multiple_of((i16[k] >> 7) * BLK, BLK)
            pltpu.async_copy(
                t_hbm.at[:, pl.ds(c, BLK)], blk_ref.at[k], sem)

    def extract(g):
        i16 = idx_ref[pl.ds(g * LANES, LANES)]
        for k in range(LANES):
            pltpu.make_async_copy(
                t_hbm.at[:, pl.ds(0, BLK)], blk_ref.at[k], sem).wait()
        for k in range(LANES):
            b = g * LANES + k
            lane = jnp.full((LANES,), i16[k] & (BLK - 1), jnp.int32)
            kfull = jnp.full((LANES,), k, jnp.int32)
            lo = plsc.load_gather(blk_ref, [kfull, lane_iota, lane])
            hi = plsc.load_gather(blk_ref, [kfull, LANES + lane_iota, lane])
            plsc.store_scatter(cols_ref, [lane_iota * BPW + b], lo)
            plsc.store_scatter(cols_ref, [(LANES + lane_iota) * BPW + b], hi)

    fire(0)

    @pl.loop(1, NGRP)
    def _(g):
        extract(g - 1)
        fire(g)

    extract(NGRP - 1)


def _dot_gather_body(uidx_hbm, vidx_hbm, ut_hbm, vt_hbm, out_hbm,
                     idx_u, idx_v, cols_u, cols_v, blk, out_v,
                     sem_u, sem_v, sem_o):
    wid = lax.axis_index("s") * NC + lax.axis_index("c")
    base = wid * BPW

    cpi_u = pltpu.async_copy(uidx_hbm.at[pl.ds(base, BPW)], idx_u, sem_u)
    cpi_v = pltpu.async_copy(vidx_hbm.at[pl.ds(base, BPW)], idx_v, sem_v)
    cpi_u.wait()
    cpi_v.wait()

    lane_iota = lax.iota(jnp.int32, LANES)

    _gather_pass(ut_hbm, idx_u, cols_u, blk, sem_u, lane_iota)
    _gather_pass(vt_hbm, idx_v, cols_v, blk, sem_v, lane_iota)

    @pl.loop(0, NGRP)
    def _(g):
        bb = g * LANES
        acc = jnp.zeros((LANES,), jnp.float32)
        for d in range(RANK):
            uu = cols_u[pl.ds(d * BPW + bb, LANES)]
            vv = cols_v[pl.ds(d * BPW + bb, LANES)]
            acc = acc + uu * vv
        out_v[pl.ds(bb, LANES)] = acc

    pltpu.async_copy(out_v, out_hbm.at[pl.ds(base, BPW)], sem_o).wait()


def kernel(idxs, U, V):
    idxs = idxs.astype(jnp.int32)
    uidx = idxs[:, 0]
    vidx = idxs[:, 1]
    mesh = plsc.VectorSubcoreMesh(core_axis_name="c", subcore_axis_name="s")
    cp = pltpu.CompilerParams()
    if "needs_layout_passes" in pltpu.CompilerParams.__dataclass_fields__:
        cp = dataclasses.replace(cp, needs_layout_passes=False)
    if "use_tc_tiling_on_sc" in pltpu.CompilerParams.__dataclass_fields__:
        cp = dataclasses.replace(cp, use_tc_tiling_on_sc=True)
    run = pl.kernel(
        _dot_gather_body,
        out_type=jax.ShapeDtypeStruct((BATCH,), jnp.float32),
        mesh=mesh,
        scratch_types=[
            pltpu.VMEM((BPW,), jnp.int32),
            pltpu.VMEM((BPW,), jnp.int32),
            pltpu.VMEM((RANK * BPW,), jnp.float32),
            pltpu.VMEM((RANK * BPW,), jnp.float32),
            pltpu.VMEM((LANES, RANK, BLK), jnp.float32),
            pltpu.VMEM((BPW,), jnp.float32),
            pltpu.SemaphoreType.DMA,
            pltpu.SemaphoreType.DMA,
            pltpu.SemaphoreType.DMA,
        ],
        compiler_params=cp,
    )
    return run(uidx, vidx, U.T, V.T)


# speedup vs baseline: 3.8221x; 1.1370x over previous
import dataclasses

import jax
import jax.numpy as jnp
from jax import lax
from jax.experimental import pallas as pl
from jax.experimental.pallas import tpu as pltpu
from jax.experimental.pallas import tpu_sc as plsc

BATCH = 16384
RANK = 32
NC = 2
NS = 16
LANES = 16
NW = NC * NS
BPW = BATCH // NW
QG = 4
NSTEP = BPW // QG
NT = BPW // LANES
BLK = 128


def _dot_gather_body(uidx_hbm, vidx_hbm, ut_hbm, vt_hbm, out_hbm,
                     idx_u, idx_v, cols_u, cols_v, blku, blkv, out_v,
                     sem_u, sem_v, sem_o):
    wid = lax.axis_index("s") * NC + lax.axis_index("c")
    base = wid * BPW

    cpu = pltpu.async_copy(uidx_hbm.at[pl.ds(base, BPW)], idx_u, sem_o)
    cpv = pltpu.async_copy(vidx_hbm.at[pl.ds(base, BPW)], idx_v, sem_o)
    cpu.wait()
    cpv.wait()

    iota = lax.iota(jnp.int32, LANES)

    def fire(t_hbm, blk, sem, i16, lane_base, p):
        for k in range(QG):
            c = pl.multiple_of((i16[lane_base + k] >> 7) * BLK, BLK)
            pltpu.async_copy(
                t_hbm.at[:, pl.ds(c, BLK)], blk.at[p, k], sem.at[p])

    def extract(t_hbm, blk, sem, cols, i16, lane_base, p, sbase):
        for k in range(QG):
            pltpu.make_async_copy(
                t_hbm.at[:, pl.ds(0, BLK)], blk.at[p, k], sem.at[p]).wait()
        pfull = jnp.full((LANES,), p, jnp.int32)
        for k in range(QG):
            b = sbase + k
            lane = jnp.full((LANES,), i16[lane_base + k] & (BLK - 1),
                            jnp.int32)
            kfull = jnp.full((LANES,), k, jnp.int32)
            lo = plsc.load_gather(blk, [pfull, kfull, iota, lane])
            hi = plsc.load_gather(blk, [pfull, kfull, LANES + iota, lane])
            plsc.store_scatter(cols, [iota * BPW + b], lo)
            plsc.store_scatter(cols, [(LANES + iota) * BPW + b], hi)

    @pl.loop(0, NT)
    def _(t):
        iu = idx_u[pl.ds(t * LANES, LANES)]
        iv = idx_v[pl.ds(t * LANES, LANES)]
        for q in range(LANES // QG):
            p = q & 1
            fire(ut_hbm, blku, sem_u, iu, q * QG, p)
            fire(vt_hbm, blkv, sem_v, iv, q * QG, p)
            sprev = t * LANES + (q - 1) * QG
            if q == 0:
                @pl.when(t > 0)
                def _():
                    iup = idx_u[pl.ds((t - 1) * LANES, LANES)]
                    ivp = idx_v[pl.ds((t - 1) * LANES, LANES)]
                    extract(ut_hbm, blku, sem_u, cols_u, iup,
                            LANES - QG, 1 - p, sprev)
                    extract(vt_hbm, blkv, sem_v, cols_v, ivp,
                            LANES - QG, 1 - p, sprev)
            else:
                extract(ut_hbm, blku, sem_u, cols_u, iu,
                        (q - 1) * QG, 1 - p, sprev)
                extract(vt_hbm, blkv, sem_v, cols_v, iv,
                        (q - 1) * QG, 1 - p, sprev)

    iul = idx_u[pl.ds((NT - 1) * LANES, LANES)]
    ivl = idx_v[pl.ds((NT - 1) * LANES, LANES)]
    extract(ut_hbm, blku, sem_u, cols_u, iul, LANES - QG, 1, BPW - QG)
    extract(vt_hbm, blkv, sem_v, cols_v, ivl, LANES - QG, 1, BPW - QG)

    @pl.loop(0, NT)
    def _(g):
        bb = g * LANES
        acc = jnp.zeros((LANES,), jnp.float32)
        for d in range(RANK):
            uu = cols_u[pl.ds(d * BPW + bb, LANES)]
            vv = cols_v[pl.ds(d * BPW + bb, LANES)]
            acc = acc + uu * vv
        out_v[pl.ds(bb, LANES)] = acc

    pltpu.async_copy(out_v, out_hbm.at[pl.ds(base, BPW)], sem_o).wait()


def kernel(idxs, U, V):
    idxs = idxs.astype(jnp.int32)
    uidx = idxs[:, 0]
    vidx = idxs[:, 1]
    mesh = plsc.VectorSubcoreMesh(core_axis_name="c", subcore_axis_name="s")
    cp = pltpu.CompilerParams()
    if "needs_layout_passes" in pltpu.CompilerParams.__dataclass_fields__:
        cp = dataclasses.replace(cp, needs_layout_passes=False)
    if "use_tc_tiling_on_sc" in pltpu.CompilerParams.__dataclass_fields__:
        cp = dataclasses.replace(cp, use_tc_tiling_on_sc=True)
    run = pl.kernel(
        _dot_gather_body,
        out_type=jax.ShapeDtypeStruct((BATCH,), jnp.float32),
        mesh=mesh,
        scratch_types=[
            pltpu.VMEM((BPW,), jnp.int32),
            pltpu.VMEM((BPW,), jnp.int32),
            pltpu.VMEM((RANK * BPW,), jnp.float32),
            pltpu.VMEM((RANK * BPW,), jnp.float32),
            pltpu.VMEM((2, QG, RANK, BLK), jnp.float32),
            pltpu.VMEM((2, QG, RANK, BLK), jnp.float32),
            pltpu.VMEM((BPW,), jnp.float32),
            pltpu.SemaphoreType.DMA((2,)),
            pltpu.SemaphoreType.DMA((2,)),
            pltpu.SemaphoreType.DMA,
        ],
        compiler_params=cp,
    )
    return run(uidx, vidx, U.T, V.T)


# speedup vs baseline: 4.1943x; 1.0974x over previous
import dataclasses

import jax
import jax.numpy as jnp
from jax import lax
from jax.experimental import pallas as pl
from jax.experimental.pallas import tpu as pltpu
from jax.experimental.pallas import tpu_sc as plsc

BATCH = 16384
RANK = 32
NC = 2
NS = 16
LANES = 16
NW = NC * NS
BPW = BATCH // NW
QG = 2
SETS = 4
LAG = 3
SPW = LANES // QG
NT = BPW // LANES
BLK = 128


def _dot_gather_body(uidx_hbm, vidx_hbm, ut_hbm, vt_hbm, out_hbm,
                     idx_u, idx_v, cols_u, cols_v, blku, blkv, out_v,
                     sem_u, sem_v, sem_o):
    wid = lax.axis_index("s") * NC + lax.axis_index("c")
    base = wid * BPW

    cpu = pltpu.async_copy(uidx_hbm.at[pl.ds(base, BPW)], idx_u, sem_o)
    cpv = pltpu.async_copy(vidx_hbm.at[pl.ds(base, BPW)], idx_v, sem_o)
    cpu.wait()
    cpv.wait()

    iota = lax.iota(jnp.int32, LANES)

    def fire(t_hbm, blk, sem, i16, lane_base, p):
        for k in range(QG):
            c = pl.multiple_of((i16[lane_base + k] >> 7) * BLK, BLK)
            pltpu.async_copy(
                t_hbm.at[:, pl.ds(c, BLK)], blk.at[p, k], sem.at[p])

    def extract(t_hbm, blk, sem, cols, i16, lane_base, p, sbase):
        for k in range(QG):
            pltpu.make_async_copy(
                t_hbm.at[:, pl.ds(0, BLK)], blk.at[p, k], sem.at[p]).wait()
        pfull = jnp.full((LANES,), p, jnp.int32)
        for k in range(QG):
            b = sbase + k
            lane = jnp.full((LANES,), i16[lane_base + k] & (BLK - 1),
                            jnp.int32)
            kfull = jnp.full((LANES,), k, jnp.int32)
            lo = plsc.load_gather(blk, [pfull, kfull, iota, lane])
            hi = plsc.load_gather(blk, [pfull, kfull, LANES + iota, lane])
            plsc.store_scatter(cols, [iota * BPW + b], lo)
            plsc.store_scatter(cols, [(LANES + iota) * BPW + b], hi)

    @pl.loop(0, NT)
    def _(t):
        iu = idx_u[pl.ds(t * LANES, LANES)]
        iv = idx_v[pl.ds(t * LANES, LANES)]
        for q in range(SPW):
            p = q & (SETS - 1)
            fire(ut_hbm, blku, sem_u, iu, q * QG, p)
            fire(vt_hbm, blkv, sem_v, iv, q * QG, p)
            qq = q - LAG
            px = qq & (SETS - 1)
            sprev = t * LANES + qq * QG
            if qq < 0:
                @pl.when(t > 0)
                def _():
                    iup = idx_u[pl.ds((t - 1) * LANES, LANES)]
                    ivp = idx_v[pl.ds((t - 1) * LANES, LANES)]
                    extract(ut_hbm, blku, sem_u, cols_u, iup,
                            (qq + SPW) * QG, px, sprev)
                    extract(vt_hbm, blkv, sem_v, cols_v, ivp,
                            (qq + SPW) * QG, px, sprev)
            else:
                extract(ut_hbm, blku, sem_u, cols_u, iu,
                        qq * QG, px, sprev)
                extract(vt_hbm, blkv, sem_v, cols_v, iv,
                        qq * QG, px, sprev)

    iul = idx_u[pl.ds((NT - 1) * LANES, LANES)]
    ivl = idx_v[pl.ds((NT - 1) * LANES, LANES)]
    for q in range(SPW - LAG, SPW):
        p = q & (SETS - 1)
        sbase = (NT - 1) * LANES + q * QG
        extract(ut_hbm, blku, sem_u, cols_u, iul, q * QG, p, sbase)
        extract(vt_hbm, blkv, sem_v, cols_v, ivl, q * QG, p, sbase)

    @pl.loop(0, NT)
    def _(g):
        bb = g * LANES
        acc = jnp.zeros((LANES,), jnp.float32)
        for d in range(RANK):
            uu = cols_u[pl.ds(d * BPW + bb, LANES)]
            vv = cols_v[pl.ds(d * BPW + bb, LANES)]
            acc = acc + uu * vv
        out_v[pl.ds(bb, LANES)] = acc

    pltpu.async_copy(out_v, out_hbm.at[pl.ds(base, BPW)], sem_o).wait()


def kernel(idxs, U, V):
    idxs = idxs.astype(jnp.int32)
    uidx = idxs[:, 0]
    vidx = idxs[:, 1]
    mesh = plsc.VectorSubcoreMesh(core_axis_name="c", subcore_axis_name="s")
    cp = pltpu.CompilerParams()
    if "needs_layout_passes" in pltpu.CompilerParams.__dataclass_fields__:
        cp = dataclasses.replace(cp, needs_layout_passes=False)
    if "use_tc_tiling_on_sc" in pltpu.CompilerParams.__dataclass_fields__:
        cp = dataclasses.replace(cp, use_tc_tiling_on_sc=True)
    run = pl.kernel(
        _dot_gather_body,
        out_type=jax.ShapeDtypeStruct((BATCH,), jnp.float32),
        mesh=mesh,
        scratch_types=[
            pltpu.VMEM((BPW,), jnp.int32),
            pltpu.VMEM((BPW,), jnp.int32),
            pltpu.VMEM((RANK * BPW,), jnp.float32),
            pltpu.VMEM((RANK * BPW,), jnp.float32),
            pltpu.VMEM((SETS, QG, RANK, BLK), jnp.float32),
            pltpu.VMEM((SETS, QG, RANK, BLK), jnp.float32),
            pltpu.VMEM((BPW,), jnp.float32),
            pltpu.SemaphoreType.DMA((SETS,)),
            pltpu.SemaphoreType.DMA((SETS,)),
            pltpu.SemaphoreType.DMA,
        ],
        compiler_params=cp,
    )
    return run(uidx, vidx, U.T, V.T)
